# Initial kernel scaffold; baseline (speedup 1.0000x reference)
import jax
import jax.numpy as jnp
from jax.experimental import pallas as pl


def kernel(x_nhwc, w1, w2, w3, g1, b1, g2, b2, g3, b3):
    raise NotImplementedError("write your pallas kernel here")


# speedup vs baseline: 1.3160x; 1.3160x over previous
import functools

import jax
import jax.numpy as jnp
from jax import lax
from jax.experimental import pallas as pl
from jax.experimental.pallas import tpu as pltpu

EPS = 1e-5
_VMEM_LIMIT = 48 * 1024 * 1024


def _round_up(x, m):
    return (x + m - 1) // m * m


def _pick_rows(mh, target=4096):
    best = 8
    for t in range(8, min(mh, target) + 1, 8):
        if mh % t == 0:
            best = t
    return best


def _compiler_params(semantics):
    return pltpu.CompilerParams(dimension_semantics=semantics,
                                vmem_limit_bytes=_VMEM_LIMIT)


def _fold(s, q, g, b, count, eps):
    mean = s * (1.0 / count)
    var = jnp.maximum(q * (1.0 / count) - mean * mean, 0.0)
    inv = lax.rsqrt(var + eps)
    scale = g * inv
    shift = b - mean * scale
    return scale, shift


def _k1_body(x_ref, w_ref, y_ref, s_ref, q_ref):
    @pl.when(pl.program_id(1) == 0)
    def _():
        s_ref[...] = jnp.zeros_like(s_ref)
        q_ref[...] = jnp.zeros_like(q_ref)

    y = jnp.dot(x_ref[...].astype(jnp.bfloat16), w_ref[...],
                preferred_element_type=jnp.float32)
    y_ref[...] = y.astype(jnp.bfloat16)
    s_ref[...] += jnp.sum(y, axis=0, keepdims=True).reshape(s_ref.shape)
    q_ref[...] += jnp.sum(y * y, axis=0, keepdims=True).reshape(q_ref.shape)


def _conv1_stats(x2d, w1b, tm, ncore):
    m, cin = x2d.shape
    cout = w1b.shape[1]
    gs = m // (ncore * tm)
    return pl.pallas_call(
        _k1_body,
        grid=(ncore, gs),
        in_specs=[pl.BlockSpec((tm, cin), lambda p, i: (p * gs + i, 0)),
                  pl.BlockSpec((cin, cout), lambda p, i: (0, 0))],
        out_specs=[pl.BlockSpec((tm, cout), lambda p, i: (p * gs + i, 0)),
                   pl.BlockSpec((1, 1, cout), lambda p, i: (p, 0, 0)),
                   pl.BlockSpec((1, 1, cout), lambda p, i: (p, 0, 0))],
        out_shape=[jax.ShapeDtypeStruct((m, cout), jnp.bfloat16),
                   jax.ShapeDtypeStruct((ncore, 1, cout), jnp.float32),
                   jax.ShapeDtypeStruct((ncore, 1, cout), jnp.float32)],
        compiler_params=_compiler_params(("parallel", "arbitrary")),
    )(x2d, w1b)


def _k2_body(y1_ref, s_ref, q_ref, g_ref, b_ref, w_ref, y2_ref, s2_ref, q2_ref,
             *, width, hw, pad_rows, count):
    @pl.when(pl.program_id(1) == 0)
    def _():
        s2_ref[...] = jnp.zeros_like(s2_ref)
        q2_ref[...] = jnp.zeros_like(q2_ref)

    s = jnp.sum(s_ref[...], axis=0)
    q = jnp.sum(q_ref[...], axis=0)
    scale, shift = _fold(s, q, g_ref[...], b_ref[...], count, EPS)
    a = jnp.maximum(y1_ref[...].astype(jnp.float32) * scale + shift, 0.0)
    ab = a.astype(jnp.bfloat16)
    cp = ab.shape[1]

    col = lax.broadcasted_iota(jnp.int32, (hw, 1), 0) % width
    ml = (col >= 1).astype(jnp.bfloat16)
    mr = (col <= width - 2).astype(jnp.bfloat16)

    zpad = jnp.zeros((pad_rows, cp), jnp.bfloat16)
    ap = jnp.concatenate([zpad, ab, zpad], axis=0)

    acc = None
    for k in range(9):
        dy, dx = k // 3 - 1, k % 3 - 1
        t = dy * width + dx
        sh = ap[pad_rows + t: pad_rows + t + hw, :]
        if dx == -1:
            sh = sh * ml
        elif dx == 1:
            sh = sh * mr
        c = jnp.dot(sh, w_ref[k], preferred_element_type=jnp.float32)
        acc = c if acc is None else acc + c

    y2_ref[...] = acc.astype(jnp.bfloat16)
    s2_ref[...] += jnp.sum(acc, axis=0, keepdims=True).reshape(s2_ref.shape)
    q2_ref[...] += jnp.sum(acc * acc, axis=0, keepdims=True).reshape(q2_ref.shape)


def _conv2_stats(y1, s1, q1, g1, b1, w2b, batch, height, width, ncore, count):
    m, cp = y1.shape
    hw = height * width
    cout = w2b.shape[2]
    pad_rows = _round_up(width + 1, 16)
    gs = batch // ncore
    kern = functools.partial(_k2_body, width=width, hw=hw, pad_rows=pad_rows,
                             count=count)
    return pl.pallas_call(
        kern,
        grid=(ncore, gs),
        in_specs=[pl.BlockSpec((hw, cp), lambda p, i: (p * gs + i, 0)),
                  pl.BlockSpec((ncore, 1, cp), lambda p, i: (0, 0, 0)),
                  pl.BlockSpec((ncore, 1, cp), lambda p, i: (0, 0, 0)),
                  pl.BlockSpec((1, cp), lambda p, i: (0, 0)),
                  pl.BlockSpec((1, cp), lambda p, i: (0, 0)),
                  pl.BlockSpec((9, cp, cout), lambda p, i: (0, 0, 0))],
        out_specs=[pl.BlockSpec((hw, cout), lambda p, i: (p * gs + i, 0)),
                   pl.BlockSpec((1, 1, cout), lambda p, i: (p, 0, 0)),
                   pl.BlockSpec((1, 1, cout), lambda p, i: (p, 0, 0))],
        out_shape=[jax.ShapeDtypeStruct((m, cout), jnp.bfloat16),
                   jax.ShapeDtypeStruct((ncore, 1, cout), jnp.float32),
                   jax.ShapeDtypeStruct((ncore, 1, cout), jnp.float32)],
        compiler_params=_compiler_params(("parallel", "arbitrary")),
    )(y1, s1, q1, g1, b1, w2b)


def _k3_body(y2_ref, s_ref, q_ref, g_ref, b_ref, a_ref, cs_ref, gram_ref,
             *, count):
    @pl.when(pl.program_id(1) == 0)
    def _():
        cs_ref[...] = jnp.zeros_like(cs_ref)
        gram_ref[...] = jnp.zeros_like(gram_ref)

    s = jnp.sum(s_ref[...], axis=0)
    q = jnp.sum(q_ref[...], axis=0)
    scale, shift = _fold(s, q, g_ref[...], b_ref[...], count, EPS)
    a = jnp.maximum(y2_ref[...].astype(jnp.float32) * scale + shift, 0.0)
    ab = a.astype(jnp.bfloat16)
    a_ref[...] = ab
    af = ab.astype(jnp.float32)
    cs_ref[...] += jnp.sum(af, axis=0, keepdims=True).reshape(cs_ref.shape)
    g = lax.dot_general(ab, ab, (((0,), (0,)), ((), ())),
                        preferred_element_type=jnp.float32)
    gram_ref[...] += g.reshape(gram_ref.shape)


def _act3_gram(y2, s2, q2, g2, b2, tm, ncore, count):
    m, cp = y2.shape
    gs = m // (ncore * tm)
    kern = functools.partial(_k3_body, count=count)
    return pl.pallas_call(
        kern,
        grid=(ncore, gs),
        in_specs=[pl.BlockSpec((tm, cp), lambda p, i: (p * gs + i, 0)),
                  pl.BlockSpec((ncore, 1, cp), lambda p, i: (0, 0, 0)),
                  pl.BlockSpec((ncore, 1, cp), lambda p, i: (0, 0, 0)),
                  pl.BlockSpec((1, cp), lambda p, i: (0, 0)),
                  pl.BlockSpec((1, cp), lambda p, i: (0, 0))],
        out_specs=[pl.BlockSpec((tm, cp), lambda p, i: (p * gs + i, 0)),
                   pl.BlockSpec((1, 1, cp), lambda p, i: (p, 0, 0)),
                   pl.BlockSpec((1, cp, cp), lambda p, i: (p, 0, 0))],
        out_shape=[jax.ShapeDtypeStruct((m, cp), jnp.bfloat16),
                   jax.ShapeDtypeStruct((ncore, 1, cp), jnp.float32),
                   jax.ShapeDtypeStruct((ncore, cp, cp), jnp.float32)],
        compiler_params=_compiler_params(("parallel", "arbitrary")),
    )(y2, s2, q2, g2, b2)


def _k4_body(a_ref, x_ref, cs_ref, gram_ref, g_ref, b_ref, w_ref, o_ref,
             *, count):
    w3 = w_ref[...]
    gram = jnp.sum(gram_ref[...], axis=0)
    cs = jnp.sum(cs_ref[...], axis=0)
    s3 = jnp.dot(cs, w3, preferred_element_type=jnp.float32)
    gw = jnp.dot(gram, w3, preferred_element_type=jnp.float32)
    q3 = jnp.sum(w3 * gw, axis=0, keepdims=True)
    scale, shift = _fold(s3, q3, g_ref[...], b_ref[...], count, EPS)
    y3 = jnp.dot(a_ref[...], w3.astype(jnp.bfloat16),
                 preferred_element_type=jnp.float32)
    o_ref[...] = jnp.maximum(y3 * scale + shift + x_ref[...], 0.0)


def _final(a2, x2d, cs, gram, g3, b3, w3, tm, ncore, count):
    m, cp = a2.shape
    c4 = w3.shape[1]
    gs = m // (ncore * tm)
    kern = functools.partial(_k4_body, count=count)
    return pl.pallas_call(
        kern,
        grid=(ncore, gs),
        in_specs=[pl.BlockSpec((tm, cp), lambda p, i: (p * gs + i, 0)),
                  pl.BlockSpec((tm, c4), lambda p, i: (p * gs + i, 0)),
                  pl.BlockSpec((ncore, 1, cp), lambda p, i: (0, 0, 0)),
                  pl.BlockSpec((ncore, cp, cp), lambda p, i: (0, 0, 0)),
                  pl.BlockSpec((1, c4), lambda p, i: (0, 0)),
                  pl.BlockSpec((1, c4), lambda p, i: (0, 0)),
                  pl.BlockSpec((cp, c4), lambda p, i: (0, 0))],
        out_specs=pl.BlockSpec((tm, c4), lambda p, i: (p * gs + i, 0)),
        out_shape=jax.ShapeDtypeStruct((m, c4), jnp.float32),
        compiler_params=_compiler_params(("parallel", "arbitrary")),
    )(a2, x2d, cs, gram, g3, b3, w3)


def kernel(x_nhwc, w1, w2, w3, g1, b1, g2, b2, g3, b3):
    n, h, w, cin = x_nhwc.shape
    m = n * h * w
    cin_pad = w1.shape[0]

    x2d = x_nhwc.reshape(m, cin)
    if cin_pad != cin:
        x2d = jnp.pad(x2d, ((0, 0), (0, cin_pad - cin)))

    ncore = 1
    tm = _pick_rows(m // ncore)
    count = float(m)

    w1b = w1.astype(jnp.bfloat16)
    w2b = w2.astype(jnp.bfloat16)

    y1, s1, q1 = _conv1_stats(x2d, w1b, tm, ncore)
    y2, s2, q2 = _conv2_stats(y1, s1, q1, g1, b1, w2b, n, h, w, ncore, count)
    a2, cs, gram = _act3_gram(y2, s2, q2, g2, b2, tm, ncore, count)
    out = _final(a2, x2d, cs, gram, g3, b3, w3, tm, ncore, count)

    if cin_pad != cin:
        out = out[:, :cin]
    return out.reshape(n, h, w, cin)


# speedup vs baseline: 1.5477x; 1.1761x over previous
import functools

import jax
import jax.numpy as jnp
from jax import lax
from jax.experimental import pallas as pl
from jax.experimental.pallas import tpu as pltpu

EPS = 1e-5
_VMEM_LIMIT = 48 * 1024 * 1024


def _round_up(x, m):
    return (x + m - 1) // m * m


def _pick_rows(mh, target=4096):
    best = 8
    for t in range(8, min(mh, target) + 1, 8):
        if mh % t == 0:
            best = t
    return best


def _compiler_params(semantics):
    return pltpu.CompilerParams(dimension_semantics=semantics,
                                vmem_limit_bytes=_VMEM_LIMIT)


def _fold(s, q, g, b, count, eps):
    mean = s * (1.0 / count)
    var = jnp.maximum(q * (1.0 / count) - mean * mean, 0.0)
    inv = lax.rsqrt(var + eps)
    scale = g * inv
    shift = b - mean * scale
    return scale, shift


def _k1_body(x_ref, w_ref, y_ref, s_ref, q_ref):
    @pl.when(pl.program_id(1) == 0)
    def _():
        s_ref[...] = jnp.zeros_like(s_ref)
        q_ref[...] = jnp.zeros_like(q_ref)

    y = jnp.dot(x_ref[...].astype(jnp.bfloat16), w_ref[...],
                preferred_element_type=jnp.float32)
    y_ref[...] = y.astype(jnp.bfloat16)
    s_ref[...] += jnp.sum(y, axis=0, keepdims=True).reshape(s_ref.shape)
    q_ref[...] += jnp.sum(y * y, axis=0, keepdims=True).reshape(q_ref.shape)


def _conv1_stats(x2d, w1b, tm, ncore):
    m, cin = x2d.shape
    cout = w1b.shape[1]
    gs = m // (ncore * tm)
    return pl.pallas_call(
        _k1_body,
        grid=(ncore, gs),
        in_specs=[pl.BlockSpec((tm, cin), lambda p, i: (p * gs + i, 0)),
                  pl.BlockSpec((cin, cout), lambda p, i: (0, 0))],
        out_specs=[pl.BlockSpec((tm, cout), lambda p, i: (p * gs + i, 0)),
                   pl.BlockSpec((1, 1, cout), lambda p, i: (p, 0, 0)),
                   pl.BlockSpec((1, 1, cout), lambda p, i: (p, 0, 0))],
        out_shape=[jax.ShapeDtypeStruct((m, cout), jnp.bfloat16),
                   jax.ShapeDtypeStruct((ncore, 1, cout), jnp.float32),
                   jax.ShapeDtypeStruct((ncore, 1, cout), jnp.float32)],
        compiler_params=_compiler_params(("parallel", "arbitrary")),
    )(x2d, w1b)


def _k2_body(y1_ref, s_ref, q_ref, g_ref, b_ref, w_ref, ml_ref, mr_ref,
             y2_ref, s2_ref, q2_ref, *, width, hw, pad_rows, count):
    @pl.when(pl.program_id(1) == 0)
    def _():
        s2_ref[...] = jnp.zeros_like(s2_ref)
        q2_ref[...] = jnp.zeros_like(q2_ref)

    s = jnp.sum(s_ref[...], axis=0)
    q = jnp.sum(q_ref[...], axis=0)
    scale, shift = _fold(s, q, g_ref[...], b_ref[...], count, EPS)
    a = jnp.maximum(y1_ref[...].astype(jnp.float32) * scale + shift, 0.0)
    ab = a.astype(jnp.bfloat16)
    cp = ab.shape[1]

    zpad = jnp.zeros((pad_rows, cp), jnp.bfloat16)
    ap = jnp.concatenate([zpad, ab, zpad], axis=0)
    lp = hw + 2 * pad_rows
    zrow = jnp.zeros((1, cp), jnp.bfloat16)
    a_l = jnp.concatenate([zrow, ap[:lp - 1]], axis=0) * ml_ref[...]
    a_r = jnp.concatenate([ap[1:], zrow], axis=0) * mr_ref[...]
    p3 = jnp.concatenate([a_l, ap, a_r], axis=1)

    c_m = jnp.dot(p3, w_ref[0], preferred_element_type=jnp.float32)
    c_0 = jnp.dot(p3, w_ref[1], preferred_element_type=jnp.float32)
    c_p = jnp.dot(p3, w_ref[2], preferred_element_type=jnp.float32)
    acc = (c_m[pad_rows - width: pad_rows - width + hw]
           + c_0[pad_rows: pad_rows + hw]
           + c_p[pad_rows + width: pad_rows + width + hw])

    y2_ref[...] = acc.astype(jnp.bfloat16)
    s2_ref[...] += jnp.sum(acc, axis=0, keepdims=True).reshape(s2_ref.shape)
    q2_ref[...] += jnp.sum(acc * acc, axis=0, keepdims=True).reshape(q2_ref.shape)


def _conv2_stats(y1, s1, q1, g1, b1, w2b, batch, height, width, ncore, count):
    m, cp = y1.shape
    hw = height * width
    cout = w2b.shape[2]
    pad_rows = _round_up(width + 1, 16)
    lp = hw + 2 * pad_rows
    gs = batch // ncore
    col = (jnp.arange(lp, dtype=jnp.int32) - pad_rows) % width
    ml = (col >= 1).astype(jnp.bfloat16).reshape(lp, 1)
    mr = (col <= width - 2).astype(jnp.bfloat16).reshape(lp, 1)
    kern = functools.partial(_k2_body, width=width, hw=hw, pad_rows=pad_rows,
                             count=count)
    return pl.pallas_call(
        kern,
        grid=(ncore, gs),
        in_specs=[pl.BlockSpec((hw, cp), lambda p, i: (p * gs + i, 0)),
                  pl.BlockSpec((ncore, 1, cp), lambda p, i: (0, 0, 0)),
                  pl.BlockSpec((ncore, 1, cp), lambda p, i: (0, 0, 0)),
                  pl.BlockSpec((1, cp), lambda p, i: (0, 0)),
                  pl.BlockSpec((1, cp), lambda p, i: (0, 0)),
                  pl.BlockSpec((3, 3 * cp, cout), lambda p, i: (0, 0, 0)),
                  pl.BlockSpec((lp, 1), lambda p, i: (0, 0)),
                  pl.BlockSpec((lp, 1), lambda p, i: (0, 0))],
        out_specs=[pl.BlockSpec((hw, cout), lambda p, i: (p * gs + i, 0)),
                   pl.BlockSpec((1, 1, cout), lambda p, i: (p, 0, 0)),
                   pl.BlockSpec((1, 1, cout), lambda p, i: (p, 0, 0))],
        out_shape=[jax.ShapeDtypeStruct((m, cout), jnp.bfloat16),
                   jax.ShapeDtypeStruct((ncore, 1, cout), jnp.float32),
                   jax.ShapeDtypeStruct((ncore, 1, cout), jnp.float32)],
        compiler_params=_compiler_params(("parallel", "arbitrary")),
    )(y1, s1, q1, g1, b1, w2b, ml, mr)


def _k3_body(y2_ref, s_ref, q_ref, g_ref, b_ref, a_ref, cs_ref, gram_ref,
             *, count):
    @pl.when(pl.program_id(1) == 0)
    def _():
        cs_ref[...] = jnp.zeros_like(cs_ref)
        gram_ref[...] = jnp.zeros_like(gram_ref)

    s = jnp.sum(s_ref[...], axis=0)
    q = jnp.sum(q_ref[...], axis=0)
    scale, shift = _fold(s, q, g_ref[...], b_ref[...], count, EPS)
    a = jnp.maximum(y2_ref[...].astype(jnp.float32) * scale + shift, 0.0)
    ab = a.astype(jnp.bfloat16)
    a_ref[...] = ab
    af = ab.astype(jnp.float32)
    cs_ref[...] += jnp.sum(af, axis=0, keepdims=True).reshape(cs_ref.shape)
    g = lax.dot_general(ab, ab, (((0,), (0,)), ((), ())),
                        preferred_element_type=jnp.float32)
    gram_ref[...] += g.reshape(gram_ref.shape)


def _act3_gram(y2, s2, q2, g2, b2, tm, ncore, count):
    m, cp = y2.shape
    gs = m // (ncore * tm)
    kern = functools.partial(_k3_body, count=count)
    return pl.pallas_call(
        kern,
        grid=(ncore, gs),
        in_specs=[pl.BlockSpec((tm, cp), lambda p, i: (p * gs + i, 0)),
                  pl.BlockSpec((ncore, 1, cp), lambda p, i: (0, 0, 0)),
                  pl.BlockSpec((ncore, 1, cp), lambda p, i: (0, 0, 0)),
                  pl.BlockSpec((1, cp), lambda p, i: (0, 0)),
                  pl.BlockSpec((1, cp), lambda p, i: (0, 0))],
        out_specs=[pl.BlockSpec((tm, cp), lambda p, i: (p * gs + i, 0)),
                   pl.BlockSpec((1, 1, cp), lambda p, i: (p, 0, 0)),
                   pl.BlockSpec((1, cp, cp), lambda p, i: (p, 0, 0))],
        out_shape=[jax.ShapeDtypeStruct((m, cp), jnp.bfloat16),
                   jax.ShapeDtypeStruct((ncore, 1, cp), jnp.float32),
                   jax.ShapeDtypeStruct((ncore, cp, cp), jnp.float32)],
        compiler_params=_compiler_params(("parallel", "arbitrary")),
    )(y2, s2, q2, g2, b2)


def _k4_body(a_ref, x_ref, cs_ref, gram_ref, g_ref, b_ref, w_ref, o_ref,
             *, count):
    w3 = w_ref[...]
    gram = jnp.sum(gram_ref[...], axis=0)
    cs = jnp.sum(cs_ref[...], axis=0)
    s3 = jnp.dot(cs, w3, preferred_element_type=jnp.float32)
    gw = jnp.dot(gram, w3, preferred_element_type=jnp.float32)
    q3 = jnp.sum(w3 * gw, axis=0, keepdims=True)
    scale, shift = _fold(s3, q3, g_ref[...], b_ref[...], count, EPS)
    y3 = jnp.dot(a_ref[...], w3.astype(jnp.bfloat16),
                 preferred_element_type=jnp.float32)
    o_ref[...] = jnp.maximum(y3 * scale + shift + x_ref[...], 0.0)


def _final(a2, x2d, cs, gram, g3, b3, w3, tm, ncore, count):
    m, cp = a2.shape
    c4 = w3.shape[1]
    gs = m // (ncore * tm)
    kern = functools.partial(_k4_body, count=count)
    return pl.pallas_call(
        kern,
        grid=(ncore, gs),
        in_specs=[pl.BlockSpec((tm, cp), lambda p, i: (p * gs + i, 0)),
                  pl.BlockSpec((tm, c4), lambda p, i: (p * gs + i, 0)),
                  pl.BlockSpec((ncore, 1, cp), lambda p, i: (0, 0, 0)),
                  pl.BlockSpec((ncore, cp, cp), lambda p, i: (0, 0, 0)),
                  pl.BlockSpec((1, c4), lambda p, i: (0, 0)),
                  pl.BlockSpec((1, c4), lambda p, i: (0, 0)),
                  pl.BlockSpec((cp, c4), lambda p, i: (0, 0))],
        out_specs=pl.BlockSpec((tm, c4), lambda p, i: (p * gs + i, 0)),
        out_shape=jax.ShapeDtypeStruct((m, c4), jnp.float32),
        compiler_params=_compiler_params(("parallel", "arbitrary")),
    )(a2, x2d, cs, gram, g3, b3, w3)


def kernel(x_nhwc, w1, w2, w3, g1, b1, g2, b2, g3, b3):
    n, h, w, cin = x_nhwc.shape
    m = n * h * w
    cin_pad = w1.shape[0]

    x2d = x_nhwc.reshape(m, cin)
    if cin_pad != cin:
        x2d = jnp.pad(x2d, ((0, 0), (0, cin_pad - cin)))

    ncore = 1
    tm = _pick_rows(m // ncore, target=8192)
    count = float(m)

    w1b = w1.astype(jnp.bfloat16)
    cp = w2.shape[1]
    w2b = w2.astype(jnp.bfloat16).reshape(3, 3 * cp, w2.shape[2])

    y1, s1, q1 = _conv1_stats(x2d, w1b, tm, ncore)
    y2, s2, q2 = _conv2_stats(y1, s1, q1, g1, b1, w2b, n, h, w, ncore, count)
    a2, cs, gram = _act3_gram(y2, s2, q2, g2, b2, tm, ncore, count)
    out = _final(a2, x2d, cs, gram, g3, b3, w3, tm, ncore, count)

    if cin_pad != cin:
        out = out[:, :cin]
    return out.reshape(n, h, w, cin)


# speedup vs baseline: 1.6627x; 1.0743x over previous
import functools

import jax
import jax.numpy as jnp
from jax import lax
from jax.experimental import pallas as pl
from jax.experimental.pallas import tpu as pltpu

EPS = 1e-5
_VMEM_LIMIT = 48 * 1024 * 1024


def _round_up(x, m):
    return (x + m - 1) // m * m


def _pick_rows(mh, target=4096):
    best = 8
    for t in range(8, min(mh, target) + 1, 8):
        if mh % t == 0:
            best = t
    return best


def _compiler_params(semantics):
    return pltpu.CompilerParams(dimension_semantics=semantics,
                                vmem_limit_bytes=_VMEM_LIMIT)


def _fold(s, q, g, b, count, eps):
    mean = s * (1.0 / count)
    var = jnp.maximum(q * (1.0 / count) - mean * mean, 0.0)
    inv = lax.rsqrt(var + eps)
    scale = g * inv
    shift = b - mean * scale
    return scale, shift


def _k1_body(x_ref, w_ref, y_ref, s_ref, q_ref):
    @pl.when(pl.program_id(1) == 0)
    def _():
        s_ref[...] = jnp.zeros_like(s_ref)
        q_ref[...] = jnp.zeros_like(q_ref)

    y = jnp.dot(x_ref[...].astype(jnp.bfloat16), w_ref[...],
                preferred_element_type=jnp.float32)
    y_ref[...] = y.astype(jnp.bfloat16)
    s_ref[...] += jnp.sum(y, axis=0, keepdims=True).reshape(s_ref.shape)
    q_ref[...] += jnp.sum(y * y, axis=0, keepdims=True).reshape(q_ref.shape)


def _conv1_stats(x2d, w1b, tm, ncore):
    m, cin = x2d.shape
    cout = w1b.shape[1]
    gs = m // (ncore * tm)
    return pl.pallas_call(
        _k1_body,
        grid=(ncore, gs),
        in_specs=[pl.BlockSpec((tm, cin), lambda p, i: (p * gs + i, 0)),
                  pl.BlockSpec((cin, cout), lambda p, i: (0, 0))],
        out_specs=[pl.BlockSpec((tm, cout), lambda p, i: (p * gs + i, 0)),
                   pl.BlockSpec((1, 1, cout), lambda p, i: (p, 0, 0)),
                   pl.BlockSpec((1, 1, cout), lambda p, i: (p, 0, 0))],
        out_shape=[jax.ShapeDtypeStruct((m, cout), jnp.bfloat16),
                   jax.ShapeDtypeStruct((ncore, 1, cout), jnp.float32),
                   jax.ShapeDtypeStruct((ncore, 1, cout), jnp.float32)],
        compiler_params=_compiler_params(("parallel", "arbitrary")),
    )(x2d, w1b)


def _k2_body(y1_ref, s_ref, q_ref, g_ref, b_ref, w_ref, ml_ref, mr_ref,
             y2_ref, s2_ref, q2_ref, *, width, hw, pad_rows, count):
    @pl.when(pl.program_id(1) == 0)
    def _():
        s2_ref[...] = jnp.zeros_like(s2_ref)
        q2_ref[...] = jnp.zeros_like(q2_ref)

    s = jnp.sum(s_ref[...], axis=0)
    q = jnp.sum(q_ref[...], axis=0)
    scale, shift = _fold(s, q, g_ref[...], b_ref[...], count, EPS)
    a = jnp.maximum(y1_ref[...].astype(jnp.float32) * scale + shift, 0.0)
    ab = a.astype(jnp.bfloat16)
    cp = ab.shape[1]

    zpad = jnp.zeros((pad_rows, cp), jnp.bfloat16)
    ap = jnp.concatenate([zpad, ab, zpad], axis=0)
    lp = hw + 2 * pad_rows
    zrow = jnp.zeros((1, cp), jnp.bfloat16)
    a_l = jnp.concatenate([zrow, ap[:lp - 1]], axis=0) * ml_ref[...]
    a_r = jnp.concatenate([ap[1:], zrow], axis=0) * mr_ref[...]
    p3 = jnp.concatenate([a_l, ap, a_r], axis=1)

    c_all = jnp.dot(p3, w_ref[...], preferred_element_type=jnp.float32)
    cout = w_ref.shape[1] // 3
    acc = (c_all[pad_rows - width: pad_rows - width + hw, 0:cout]
           + c_all[pad_rows: pad_rows + hw, cout:2 * cout]
           + c_all[pad_rows + width: pad_rows + width + hw, 2 * cout:3 * cout])

    y2_ref[...] = acc.astype(jnp.bfloat16)
    s2_ref[...] += jnp.sum(acc, axis=0, keepdims=True).reshape(s2_ref.shape)
    q2_ref[...] += jnp.sum(acc * acc, axis=0, keepdims=True).reshape(q2_ref.shape)


def _conv2_stats(y1, s1, q1, g1, b1, w2b, batch, height, width, ncore, count):
    m, cp = y1.shape
    hw = height * width
    cout = w2b.shape[1] // 3
    pad_rows = _round_up(width + 1, 16)
    lp = hw + 2 * pad_rows
    gs = batch // ncore
    col = (jnp.arange(lp, dtype=jnp.int32) - pad_rows) % width
    ml = (col >= 1).astype(jnp.bfloat16).reshape(lp, 1)
    mr = (col <= width - 2).astype(jnp.bfloat16).reshape(lp, 1)
    kern = functools.partial(_k2_body, width=width, hw=hw, pad_rows=pad_rows,
                             count=count)
    return pl.pallas_call(
        kern,
        grid=(ncore, gs),
        in_specs=[pl.BlockSpec((hw, cp), lambda p, i: (p * gs + i, 0)),
                  pl.BlockSpec((ncore, 1, cp), lambda p, i: (0, 0, 0)),
                  pl.BlockSpec((ncore, 1, cp), lambda p, i: (0, 0, 0)),
                  pl.BlockSpec((1, cp), lambda p, i: (0, 0)),
                  pl.BlockSpec((1, cp), lambda p, i: (0, 0)),
                  pl.BlockSpec((3 * cp, 3 * cout), lambda p, i: (0, 0)),
                  pl.BlockSpec((lp, 1), lambda p, i: (0, 0)),
                  pl.BlockSpec((lp, 1), lambda p, i: (0, 0))],
        out_specs=[pl.BlockSpec((hw, cout), lambda p, i: (p * gs + i, 0)),
                   pl.BlockSpec((1, 1, cout), lambda p, i: (p, 0, 0)),
                   pl.BlockSpec((1, 1, cout), lambda p, i: (p, 0, 0))],
        out_shape=[jax.ShapeDtypeStruct((m, cout), jnp.bfloat16),
                   jax.ShapeDtypeStruct((ncore, 1, cout), jnp.float32),
                   jax.ShapeDtypeStruct((ncore, 1, cout), jnp.float32)],
        compiler_params=_compiler_params(("parallel", "arbitrary")),
    )(y1, s1, q1, g1, b1, w2b, ml, mr)


def _k3_body(y2_ref, s_ref, q_ref, g_ref, b_ref, cs_ref, gram_ref,
             *, count):
    @pl.when(pl.program_id(1) == 0)
    def _():
        cs_ref[...] = jnp.zeros_like(cs_ref)
        gram_ref[...] = jnp.zeros_like(gram_ref)

    s = jnp.sum(s_ref[...], axis=0)
    q = jnp.sum(q_ref[...], axis=0)
    scale, shift = _fold(s, q, g_ref[...], b_ref[...], count, EPS)
    a = jnp.maximum(y2_ref[...].astype(jnp.float32) * scale + shift, 0.0)
    ab = a.astype(jnp.bfloat16)
    af = ab.astype(jnp.float32)
    cs_ref[...] += jnp.sum(af, axis=0, keepdims=True).reshape(cs_ref.shape)
    g = lax.dot_general(ab, ab, (((0,), (0,)), ((), ())),
                        preferred_element_type=jnp.float32)
    gram_ref[...] += g.reshape(gram_ref.shape)


def _act3_gram(y2, s2, q2, g2, b2, tm, ncore, count):
    m, cp = y2.shape
    gs = m // (ncore * tm)
    kern = functools.partial(_k3_body, count=count)
    return pl.pallas_call(
        kern,
        grid=(ncore, gs),
        in_specs=[pl.BlockSpec((tm, cp), lambda p, i: (p * gs + i, 0)),
                  pl.BlockSpec((ncore, 1, cp), lambda p, i: (0, 0, 0)),
                  pl.BlockSpec((ncore, 1, cp), lambda p, i: (0, 0, 0)),
                  pl.BlockSpec((1, cp), lambda p, i: (0, 0)),
                  pl.BlockSpec((1, cp), lambda p, i: (0, 0))],
        out_specs=[pl.BlockSpec((1, 1, cp), lambda p, i: (p, 0, 0)),
                   pl.BlockSpec((1, cp, cp), lambda p, i: (p, 0, 0))],
        out_shape=[jax.ShapeDtypeStruct((ncore, 1, cp), jnp.float32),
                   jax.ShapeDtypeStruct((ncore, cp, cp), jnp.float32)],
        compiler_params=_compiler_params(("parallel", "arbitrary")),
    )(y2, s2, q2, g2, b2)


def _k4_body(y2_ref, x_ref, s_ref, q_ref, g2_ref, b2_ref, cs_ref, gram_ref,
             g3_ref, b3_ref, w_ref, o_ref, *, count):
    s = jnp.sum(s_ref[...], axis=0)
    q = jnp.sum(q_ref[...], axis=0)
    sc2, sh2 = _fold(s, q, g2_ref[...], b2_ref[...], count, EPS)
    a2 = jnp.maximum(y2_ref[...].astype(jnp.float32) * sc2 + sh2, 0.0)
    ab = a2.astype(jnp.bfloat16)

    w3 = w_ref[...]
    gram = jnp.sum(gram_ref[...], axis=0)
    cs = jnp.sum(cs_ref[...], axis=0)
    s3 = jnp.dot(cs, w3, preferred_element_type=jnp.float32)
    gw = jnp.dot(gram, w3, preferred_element_type=jnp.float32)
    q3 = jnp.sum(w3 * gw, axis=0, keepdims=True)
    scale, shift = _fold(s3, q3, g3_ref[...], b3_ref[...], count, EPS)
    y3 = jnp.dot(ab, w3.astype(jnp.bfloat16),
                 preferred_element_type=jnp.float32)
    o_ref[...] = jnp.maximum(y3 * scale + shift + x_ref[...], 0.0)


def _final(y2, x2d, s2, q2, g2, b2, cs, gram, g3, b3, w3, tm, ncore, count):
    m, cp = y2.shape
    c4 = w3.shape[1]
    gs = m // (ncore * tm)
    kern = functools.partial(_k4_body, count=count)
    return pl.pallas_call(
        kern,
        grid=(ncore, gs),
        in_specs=[pl.BlockSpec((tm, cp), lambda p, i: (p * gs + i, 0)),
                  pl.BlockSpec((tm, c4), lambda p, i: (p * gs + i, 0)),
                  pl.BlockSpec((ncore, 1, cp), lambda p, i: (0, 0, 0)),
                  pl.BlockSpec((ncore, 1, cp), lambda p, i: (0, 0, 0)),
                  pl.BlockSpec((1, cp), lambda p, i: (0, 0)),
                  pl.BlockSpec((1, cp), lambda p, i: (0, 0)),
                  pl.BlockSpec((ncore, 1, cp), lambda p, i: (0, 0, 0)),
                  pl.BlockSpec((ncore, cp, cp), lambda p, i: (0, 0, 0)),
                  pl.BlockSpec((1, c4), lambda p, i: (0, 0)),
                  pl.BlockSpec((1, c4), lambda p, i: (0, 0)),
                  pl.BlockSpec((cp, c4), lambda p, i: (0, 0))],
        out_specs=pl.BlockSpec((tm, c4), lambda p, i: (p * gs + i, 0)),
        out_shape=jax.ShapeDtypeStruct((m, c4), jnp.float32),
        compiler_params=_compiler_params(("parallel", "arbitrary")),
    )(y2, x2d, s2, q2, g2, b2, cs, gram, g3, b3, w3)


def kernel(x_nhwc, w1, w2, w3, g1, b1, g2, b2, g3, b3):
    n, h, w, cin = x_nhwc.shape
    m = n * h * w
    cin_pad = w1.shape[0]

    x2d = x_nhwc.reshape(m, cin)
    if cin_pad != cin:
        x2d = jnp.pad(x2d, ((0, 0), (0, cin_pad - cin)))

    ncore = 1
    tm = _pick_rows(m // ncore, target=8192)
    count = float(m)

    w1b = w1.astype(jnp.bfloat16)
    cp = w2.shape[1]
    cout2 = w2.shape[2]
    w2b = (w2.astype(jnp.bfloat16).reshape(3, 3, cp, cout2)
           .transpose(1, 2, 0, 3).reshape(3 * cp, 3 * cout2))

    y1, s1, q1 = _conv1_stats(x2d, w1b, tm, ncore)
    y2, s2, q2 = _conv2_stats(y1, s1, q1, g1, b1, w2b, n, h, w, ncore, count)
    cs, gram = _act3_gram(y2, s2, q2, g2, b2, tm, ncore, count)
    out = _final(y2, x2d, s2, q2, g2, b2, cs, gram, g3, b3, w3, tm, ncore,
                 count)

    if cin_pad != cin:
        out = out[:, :cin]
    return out.reshape(n, h, w, cin)


# speedup vs baseline: 1.9945x; 1.1996x over previous
import functools

import jax
import jax.numpy as jnp
from jax import lax
from jax.experimental import pallas as pl
from jax.experimental.pallas import tpu as pltpu

EPS = 1e-5
_VMEM_LIMIT = 56 * 1024 * 1024


def _round_up(x, m):
    return (x + m - 1) // m * m


def _fold(s, q, g, b, count, eps):
    mean = s * (1.0 / count)
    var = jnp.maximum(q * (1.0 / count) - mean * mean, 0.0)
    inv = lax.rsqrt(var + eps)
    scale = g * inv
    shift = b - mean * scale
    return scale, shift


def _mega_body(x_ref, w1_ref, w2_ref, w3_ref, g1_ref, b1_ref, g2_ref, b2_ref,
               g3_ref, b3_ref, ml_ref, mr_ref, o_ref,
               xb_s, y1_s, y2_s, s1_s, q1_s, s2_s, q2_s, cs_s, gram_s,
               *, gs_a, n_img, n_c, gs_d, tm, tc, hw, width, pad_rows, count):
    i = pl.program_id(0)

    @pl.when(i == 0)
    def _():
        s1_s[...] = jnp.zeros_like(s1_s)
        q1_s[...] = jnp.zeros_like(q1_s)
        s2_s[...] = jnp.zeros_like(s2_s)
        q2_s[...] = jnp.zeros_like(q2_s)
        cs_s[...] = jnp.zeros_like(cs_s)
        gram_s[...] = jnp.zeros_like(gram_s)

    @pl.when(i < gs_a)
    def _():
        x = x_ref[...]
        xb = x.astype(jnp.bfloat16)
        row = pl.multiple_of(i * tm, tm)
        xb_s[pl.ds(row, tm), :] = xb
        y = jnp.dot(xb, w1_ref[...], preferred_element_type=jnp.float32)
        y1_s[pl.ds(row, tm), :] = y.astype(jnp.bfloat16)
        s1_s[...] += jnp.sum(y, axis=0, keepdims=True)
        q1_s[...] += jnp.sum(y * y, axis=0, keepdims=True)

    @pl.when((i >= gs_a) & (i < gs_a + n_img))
    def _():
        img = i - gs_a
        scale, shift = _fold(s1_s[...], q1_s[...], g1_ref[...], b1_ref[...],
                             count, EPS)
        row = pl.multiple_of(img * hw, hw)
        yb = y1_s[pl.ds(row, hw), :]
        a = jnp.maximum(yb.astype(jnp.float32) * scale + shift, 0.0)
        ab = a.astype(jnp.bfloat16)
        cp = ab.shape[1]

        zpad = jnp.zeros((pad_rows, cp), jnp.bfloat16)
        ap = jnp.concatenate([zpad, ab, zpad], axis=0)
        lp = hw + 2 * pad_rows
        zrow = jnp.zeros((1, cp), jnp.bfloat16)
        a_l = jnp.concatenate([zrow, ap[:lp - 1]], axis=0) * ml_ref[...]
        a_r = jnp.concatenate([ap[1:], zrow], axis=0) * mr_ref[...]
        p3 = jnp.concatenate([a_l, ap, a_r], axis=1)

        c_all = jnp.dot(p3, w2_ref[...], preferred_element_type=jnp.float32)
        cout = w2_ref.shape[1] // 3
        acc = (c_all[pad_rows - width: pad_rows - width + hw, 0:cout]
               + c_all[pad_rows: pad_rows + hw, cout:2 * cout]
               + c_all[pad_rows + width: pad_rows + width + hw,
                       2 * cout:3 * cout])

        y2_s[pl.ds(row, hw), :] = acc.astype(jnp.bfloat16)
        s2_s[...] += jnp.sum(acc, axis=0, keepdims=True)
        q2_s[...] += jnp.sum(acc * acc, axis=0, keepdims=True)

    @pl.when((i >= gs_a + n_img) & (i < gs_a + n_img + n_c))
    def _():
        j = i - gs_a - n_img
        scale, shift = _fold(s2_s[...], q2_s[...], g2_ref[...], b2_ref[...],
                             count, EPS)
        row = pl.multiple_of(j * tc, tc)
        yb = y2_s[pl.ds(row, tc), :]
        a2 = jnp.maximum(yb.astype(jnp.float32) * scale + shift, 0.0)
        ab = a2.astype(jnp.bfloat16)
        cs_s[...] += jnp.sum(a2, axis=0, keepdims=True)
        gram_s[...] += lax.dot_general(ab, ab, (((0,), (0,)), ((), ())),
                                       preferred_element_type=jnp.float32)

    @pl.when(i >= gs_a + n_img + n_c)
    def _():
        j = i - gs_a - n_img - n_c
        sc2, sh2 = _fold(s2_s[...], q2_s[...], g2_ref[...], b2_ref[...],
                         count, EPS)
        row = pl.multiple_of(j * tm, tm)
        yb = y2_s[pl.ds(row, tm), :]
        a2 = jnp.maximum(yb.astype(jnp.float32) * sc2 + sh2, 0.0)
        ab = a2.astype(jnp.bfloat16)

        w3 = w3_ref[...]
        s3 = jnp.dot(cs_s[...], w3, preferred_element_type=jnp.float32)
        gw = jnp.dot(gram_s[...], w3, preferred_element_type=jnp.float32)
        q3 = jnp.sum(w3 * gw, axis=0, keepdims=True)
        sc3, sh3 = _fold(s3, q3, g3_ref[...], b3_ref[...], count, EPS)

        y3 = jnp.dot(ab, w3.astype(jnp.bfloat16),
                     preferred_element_type=jnp.float32)
        xres = xb_s[pl.ds(row, tm), :].astype(jnp.float32)
        o_ref[...] = jnp.maximum(y3 * sc3 + sh3 + xres, 0.0)


def kernel(x_nhwc, w1, w2, w3, g1, b1, g2, b2, g3, b3):
    n, h, w, cin = x_nhwc.shape
    m = n * h * w
    hw = h * w
    cin_pad = w1.shape[0]

    x2d = x_nhwc.reshape(m, cin)
    if cin_pad != cin:
        x2d = jnp.pad(x2d, ((0, 0), (0, cin_pad - cin)))

    cp = w2.shape[1]
    cout2 = w2.shape[2]
    c4 = w3.shape[1]
    count = float(m)

    tm = hw
    gs_a = m // tm
    n_img = n
    n_c = 2 if (m // 2) % 8 == 0 else 1
    tc = m // n_c
    gs_d = m // tm
    pad_rows = _round_up(w + 1, 16)
    lp = hw + 2 * pad_rows
    grid = gs_a + n_img + n_c + gs_d

    w1b = w1.astype(jnp.bfloat16)
    w2b = (w2.astype(jnp.bfloat16).reshape(3, 3, cp, cout2)
           .transpose(1, 2, 0, 3).reshape(3 * cp, 3 * cout2))

    col = (jnp.arange(lp, dtype=jnp.int32) - pad_rows) % w
    ml = (col >= 1).astype(jnp.bfloat16).reshape(lp, 1)
    mr = (col <= w - 2).astype(jnp.bfloat16).reshape(lp, 1)

    kern = functools.partial(
        _mega_body, gs_a=gs_a, n_img=n_img, n_c=n_c, gs_d=gs_d,
        tm=tm, tc=tc, hw=hw, width=w, pad_rows=pad_rows, count=count)

    def _x_map(i):
        return (jnp.minimum(i, gs_a - 1), 0)

    def _o_map(i):
        return (jnp.maximum(i - (gs_a + n_img + n_c), 0), 0)

    out = pl.pallas_call(
        kern,
        grid=(grid,),
        in_specs=[pl.BlockSpec((tm, cin_pad), _x_map),
                  pl.BlockSpec((cin_pad, cp), lambda i: (0, 0)),
                  pl.BlockSpec((3 * cp, 3 * cout2), lambda i: (0, 0)),
                  pl.BlockSpec((cp, c4), lambda i: (0, 0)),
                  pl.BlockSpec((1, cp), lambda i: (0, 0)),
                  pl.BlockSpec((1, cp), lambda i: (0, 0)),
                  pl.BlockSpec((1, cp), lambda i: (0, 0)),
                  pl.BlockSpec((1, cp), lambda i: (0, 0)),
                  pl.BlockSpec((1, c4), lambda i: (0, 0)),
                  pl.BlockSpec((1, c4), lambda i: (0, 0)),
                  pl.BlockSpec((lp, 1), lambda i: (0, 0)),
                  pl.BlockSpec((lp, 1), lambda i: (0, 0))],
        out_specs=pl.BlockSpec((tm, c4), _o_map),
        out_shape=jax.ShapeDtypeStruct((m, c4), jnp.float32),
        scratch_shapes=[pltpu.VMEM((m, cin_pad), jnp.bfloat16),
                        pltpu.VMEM((m, cp), jnp.bfloat16),
                        pltpu.VMEM((m, cout2), jnp.bfloat16),
                        pltpu.VMEM((1, cp), jnp.float32),
                        pltpu.VMEM((1, cp), jnp.float32),
                        pltpu.VMEM((1, cout2), jnp.float32),
                        pltpu.VMEM((1, cout2), jnp.float32),
                        pltpu.VMEM((1, cout2), jnp.float32),
                        pltpu.VMEM((cout2, cout2), jnp.float32)],
        compiler_params=pltpu.CompilerParams(
            dimension_semantics=("arbitrary",),
            vmem_limit_bytes=_VMEM_LIMIT),
    )(x2d, w1b, w2b, w3, g1, b1, g2, b2, g3, b3, ml, mr)

    if cin_pad != cin:
        out = out[:, :cin]
    return out.reshape(n, h, w, cin)
